# Initial kernel scaffold; baseline (speedup 1.0000x reference)
import jax
import jax.numpy as jnp
from jax.experimental import pallas as pl


def kernel(x0, data, t, condition_mask):
    raise NotImplementedError("write your pallas kernel here")


# speedup vs baseline: 1.6803x; 1.6803x over previous
import functools

import jax
import jax.numpy as jnp
from jax import lax
from jax.experimental import pallas as pl

_B, _S, _D = 4, 2048, 16
_Q = 512


def _fm_kernel(t_ref, mask_ref, x0_ref, data_ref, out_ref):
    qi = pl.program_id(1)
    t = t_ref[0, 0, 0]
    mask = mask_ref[0] > 0.0

    dat = data_ref[0]
    x0 = x0_ref[0]
    dat_q = data_ref[0, pl.ds(qi * _Q, _Q), :]

    x = jnp.where(mask, dat_q, x0)

    xn = jnp.sum(x * x, axis=1, keepdims=True)
    dn = jnp.sum(dat * dat, axis=1)[None, :]
    xy = lax.dot_general(
        x, dat, (((1,), (1,)), ((), ())),
        preferred_element_type=jnp.float32,
        precision=lax.Precision.HIGHEST,
    )
    d2 = (xn - 2.0 * xy) + dn
    idx = jnp.argmin(d2, axis=1)

    onehot = (lax.broadcasted_iota(jnp.int32, (_Q, _S), 1) == idx[:, None])
    nearest = lax.dot_general(
        onehot.astype(jnp.float32), dat, (((1,), (0,)), ((), ())),
        preferred_element_type=jnp.float32,
        precision=lax.Precision.HIGHEST,
    )

    xt = jnp.where(mask, dat_q, t * dat_q + (1.0 - t) * x)
    ut = dat_q - x
    out_ref[0] = jnp.concatenate([xt, ut, nearest], axis=1)


@jax.jit
def kernel(x0, data, t, condition_mask):
    t3 = t.reshape(_B, 1, 1).astype(jnp.float32)
    mask_f = condition_mask.reshape(1, _D).astype(jnp.float32)
    out = pl.pallas_call(
        _fm_kernel,
        grid=(_B, _S // _Q),
        in_specs=[
            pl.BlockSpec((1, 1, 1), lambda b, q: (b, 0, 0)),
            pl.BlockSpec((1, _D), lambda b, q: (0, 0)),
            pl.BlockSpec((1, _Q, _D), lambda b, q: (b, q, 0)),
            pl.BlockSpec((1, _S, _D), lambda b, q: (b, 0, 0)),
        ],
        out_specs=pl.BlockSpec((1, _Q, 3 * _D), lambda b, q: (b, q, 0)),
        out_shape=jax.ShapeDtypeStruct((_B, _S, 3 * _D), jnp.float32),
    )(t3, mask_f, x0, data)
    return out


# speedup vs baseline: 3.6462x; 2.1700x over previous
import functools

import jax
import jax.numpy as jnp
from jax import lax
from jax.experimental import pallas as pl
from jax.experimental.pallas import tpu as pltpu
from jax.experimental.pallas import tpu_sc as plsc

_B, _S, _D = 4, 2048, 16
_Q = 512

_NC, _NS = 2, 16
_NW = _NC * _NS
_ROWS = _B * _S
_RPW = _ROWS // _NW
_CHUNK = 128


def _fm_kernel(t_ref, mask_ref, x0_ref, data_ref, out_ref, idx_ref):
    b = pl.program_id(0)
    qi = pl.program_id(1)
    t = t_ref[0, 0, 0]
    mask = mask_ref[0] > 0.0

    dat = data_ref[0]
    x0 = x0_ref[0]
    dat_q = data_ref[0, pl.ds(qi * _Q, _Q), :]

    x = jnp.where(mask, dat_q, x0)

    xn = jnp.sum(x * x, axis=1, keepdims=True)
    dn = jnp.sum(dat * dat, axis=1)[None, :]
    xy = lax.dot_general(
        x, dat, (((1,), (1,)), ((), ())),
        preferred_element_type=jnp.float32,
        precision=lax.Precision.HIGHEST,
    )
    d2 = (xn - 2.0 * xy) + dn
    idx = jnp.argmin(d2, axis=1)
    idx_ref[0, 0, :] = idx.astype(jnp.int32) + b * _S

    xt = jnp.where(mask, dat_q, t * dat_q + (1.0 - t) * x)
    ut = dat_q - x
    out_ref[0] = jnp.concatenate([xt, ut], axis=1)


_sc_mesh = plsc.VectorSubcoreMesh(
    core_axis_name="c", subcore_axis_name="s",
    num_cores=_NC, num_subcores=_NS,
)


@functools.partial(
    pl.kernel,
    out_type=jax.ShapeDtypeStruct((_ROWS, _D), jnp.float32),
    mesh=_sc_mesh,
    scratch_types=[
        pltpu.VMEM((_RPW,), jnp.int32),
        pltpu.VMEM((_RPW, _D), jnp.float32),
        pltpu.SemaphoreType.DMA,
    ],
    compiler_params=pltpu.CompilerParams(use_tc_tiling_on_sc=False),
)
def _gather_kernel(data_hbm, idx_hbm, out_hbm, idx_v, rows_v, sem):
    wid = lax.axis_index("s") * _NC + lax.axis_index("c")
    base = wid * _RPW
    pltpu.sync_copy(idx_hbm.at[pl.ds(base, _RPW)], idx_v)
    copies = [
        pltpu.async_copy(
            data_hbm.at[idx_v.at[pl.ds(j * _CHUNK, _CHUNK)]],
            rows_v.at[pl.ds(j * _CHUNK, _CHUNK)],
            sem,
        )
        for j in range(_RPW // _CHUNK)
    ]
    for c in copies:
        c.wait()
    pltpu.sync_copy(rows_v, out_hbm.at[pl.ds(base, _RPW)])


@jax.jit
def kernel(x0, data, t, condition_mask):
    t3 = t.reshape(_B, 1, 1).astype(jnp.float32)
    mask_f = condition_mask.reshape(1, _D).astype(jnp.float32)
    out32, idxg = pl.pallas_call(
        _fm_kernel,
        grid=(_B, _S // _Q),
        in_specs=[
            pl.BlockSpec((1, 1, 1), lambda b, q: (b, 0, 0)),
            pl.BlockSpec((1, _D), lambda b, q: (0, 0)),
            pl.BlockSpec((1, _Q, _D), lambda b, q: (b, q, 0)),
            pl.BlockSpec((1, _S, _D), lambda b, q: (b, 0, 0)),
        ],
        out_specs=[
            pl.BlockSpec((1, _Q, 2 * _D), lambda b, q: (b, q, 0)),
            pl.BlockSpec((1, 1, _Q), lambda b, q: (b, 0, q)),
        ],
        out_shape=[
            jax.ShapeDtypeStruct((_B, _S, 2 * _D), jnp.float32),
            jax.ShapeDtypeStruct((_B, 1, _S), jnp.int32),
        ],
    )(t3, mask_f, x0, data)
    nearest = _gather_kernel(data.reshape(_ROWS, _D), idxg.reshape(_ROWS))
    return jnp.concatenate([out32, nearest.reshape(_B, _S, _D)], axis=-1)


# speedup vs baseline: 4.8223x; 1.3225x over previous
import functools

import jax
import jax.numpy as jnp
from jax import lax
from jax.experimental import pallas as pl
from jax.experimental.pallas import tpu as pltpu
from jax.experimental.pallas import tpu_sc as plsc

_B, _S, _D = 4, 2048, 16
_Q = 2048
_CS = 512
_NCH = _S // _CS

_NC, _NS = 2, 16
_NW = _NC * _NS
_ROWS = _B * _S
_RPW = _ROWS // _NW
_CHUNK = 128


def _fm_kernel(mask_ref, data_ref, x0h_ref, x0m_ref, x0l_ref,
               yh_ref, ym_ref, yl_ref, idx_ref):
    b = pl.program_id(0)
    mask = mask_ref[0] > 0.0

    dat = data_ref[0]
    yh, ym, yl = yh_ref[0], ym_ref[0], yl_ref[0]
    xh = jnp.where(mask, yh, x0h_ref[0])
    xm = jnp.where(mask, ym, x0m_ref[0])
    xl = jnp.where(mask, yl, x0l_ref[0])

    dn = jnp.sum(dat * dat, axis=1, keepdims=True)
    hdn = -0.5 * dn
    nh = hdn.astype(jnp.bfloat16)
    r1 = hdn - nh.astype(jnp.float32)
    nm = r1.astype(jnp.bfloat16)
    nl = (r1 - nm.astype(jnp.float32)).astype(jnp.bfloat16)

    ones = jnp.ones((_Q, 3), jnp.bfloat16)
    lhs = jnp.concatenate([xh, xm, xl, xh, xm, xh, ones], axis=1)
    rhs = jnp.concatenate([yh, yh, yh, ym, ym, yl, nh, nm, nl], axis=1)

    lane = lax.broadcasted_iota(jnp.int32, (_Q, _CS), 1)
    run_v = jnp.zeros((_Q, _CS), jnp.float32)
    run_col = lane
    for c in range(_NCH):
        s = lax.dot_general(
            lhs, rhs[c * _CS:(c + 1) * _CS],
            (((1,), (1,)), ((), ())),
            preferred_element_type=jnp.float32,
        )
        if c == 0:
            run_v = s
        else:
            gt = s > run_v
            run_v = jnp.where(gt, s, run_v)
            run_col = jnp.where(gt, lane + c * _CS, run_col)
    m = jnp.max(run_v, axis=1, keepdims=True)
    cand = jnp.where(run_v == m, run_col, jnp.int32(_S))
    idx = jnp.min(cand, axis=1)
    idx_ref[0, 0, :] = idx + b * _S


_sc_mesh = plsc.VectorSubcoreMesh(
    core_axis_name="c", subcore_axis_name="s",
    num_cores=_NC, num_subcores=_NS,
)


@functools.partial(
    pl.kernel,
    out_type=jax.ShapeDtypeStruct((_ROWS, 3 * _D), jnp.float32),
    mesh=_sc_mesh,
    scratch_types=[
        pltpu.VMEM((_RPW,), jnp.int32),
        pltpu.VMEM((_RPW, _D), jnp.float32),
        pltpu.VMEM((_RPW, _D), jnp.float32),
        pltpu.VMEM((_RPW, _D), jnp.float32),
        pltpu.VMEM((_RPW, 3 * _D), jnp.float32),
        pltpu.VMEM((_D,), jnp.float32),
        pltpu.VMEM((_D,), jnp.float32),
        pltpu.SemaphoreType.DMA,
    ],
    compiler_params=pltpu.CompilerParams(use_tc_tiling_on_sc=False),
)
def _sc_kernel(x0_hbm, data_hbm, t_hbm, mask_hbm, idx_hbm, out_hbm,
               idx_v, x0_v, dat_v, near_v, out_v, t_v, m_v, sem):
    wid = lax.axis_index("s") * _NC + lax.axis_index("c")
    base = wid * _RPW
    bidx = wid // (_NW // _B)
    pltpu.sync_copy(idx_hbm.at[pl.ds(base, _RPW)], idx_v)
    copies = [
        pltpu.async_copy(
            data_hbm.at[idx_v.at[pl.ds(j * _CHUNK, _CHUNK)]],
            near_v.at[pl.ds(j * _CHUNK, _CHUNK)],
            sem,
        )
        for j in range(_RPW // _CHUNK)
    ]
    pltpu.sync_copy(x0_hbm.at[pl.ds(base, _RPW)], x0_v)
    pltpu.sync_copy(data_hbm.at[pl.ds(base, _RPW)], dat_v)
    pltpu.sync_copy(t_hbm.at[bidx], t_v)
    pltpu.sync_copy(mask_hbm.at[0], m_v)
    for cpy in copies:
        cpy.wait()

    tv = t_v[...]
    mv = m_v[...] > 0.0
    omt = 1.0 - tv

    def body(i, carry):
        d = dat_v[i, :]
        x0r = x0_v[i, :]
        xmg = jnp.where(mv, d, x0r)
        out_v[i, pl.ds(0, _D)] = jnp.where(mv, d, tv * d + omt * xmg)
        out_v[i, pl.ds(_D, _D)] = d - xmg
        out_v[i, pl.ds(2 * _D, _D)] = near_v[i, :]
        return carry

    lax.fori_loop(0, _RPW, body, 0)
    pltpu.sync_copy(out_v, out_hbm.at[pl.ds(base, _RPW)])


def _split3(v):
    hi = v.astype(jnp.bfloat16)
    r1 = v - hi.astype(jnp.float32)
    mid = r1.astype(jnp.bfloat16)
    lo = (r1 - mid.astype(jnp.float32)).astype(jnp.bfloat16)
    return hi, mid, lo


@jax.jit
def kernel(x0, data, t, condition_mask):
    mask_f = condition_mask.reshape(1, _D).astype(jnp.float32)
    x0h, x0m, x0l = _split3(x0)
    yh, ym, yl = _split3(data)
    idxg = pl.pallas_call(
        _fm_kernel,
        grid=(_B,),
        in_specs=[
            pl.BlockSpec((1, _D), lambda b: (0, 0)),
            pl.BlockSpec((1, _S, _D), lambda b: (b, 0, 0)),
            pl.BlockSpec((1, _Q, _D), lambda b: (b, 0, 0)),
            pl.BlockSpec((1, _Q, _D), lambda b: (b, 0, 0)),
            pl.BlockSpec((1, _Q, _D), lambda b: (b, 0, 0)),
            pl.BlockSpec((1, _S, _D), lambda b: (b, 0, 0)),
            pl.BlockSpec((1, _S, _D), lambda b: (b, 0, 0)),
            pl.BlockSpec((1, _S, _D), lambda b: (b, 0, 0)),
        ],
        out_specs=pl.BlockSpec((1, 1, _Q), lambda b: (b, 0, 0)),
        out_shape=jax.ShapeDtypeStruct((_B, 1, _S), jnp.int32),
    )(mask_f, data, x0h, x0m, x0l, yh, ym, yl)
    t4 = jnp.broadcast_to(t.reshape(_B, 1), (_B, _D)).astype(jnp.float32)
    out = _sc_kernel(
        x0.reshape(_ROWS, _D),
        data.reshape(_ROWS, _D),
        t4,
        mask_f,
        idxg.reshape(_ROWS),
    )
    return out.reshape(_B, _S, 3 * _D)


# speedup vs baseline: 4.9393x; 1.0243x over previous
import functools

import jax
import jax.numpy as jnp
from jax import lax
from jax.experimental import pallas as pl
from jax.experimental.pallas import tpu as pltpu
from jax.experimental.pallas import tpu_sc as plsc

_B, _S, _D = 4, 2048, 16
_Q = 2048
_CS = 512
_NCH = _S // _CS

_NC, _NS = 2, 16
_NW = _NC * _NS
_ROWS = _B * _S
_RPW = _ROWS // _NW
_CHUNK = 128


def _fm_kernel(mask_ref, data_ref, x0_ref, idx_ref):
    b = pl.program_id(0)
    mask = mask_ref[0] > 0.0

    dat = data_ref[0]
    x = jnp.where(mask, dat, x0_ref[0])
    yh, ym, yl = _split3(dat)
    xh, xm, xl = _split3(x)

    dn = jnp.sum(dat * dat, axis=1, keepdims=True)
    hdn = -0.5 * dn
    nh = hdn.astype(jnp.bfloat16)
    r1 = hdn - nh.astype(jnp.float32)
    nm = r1.astype(jnp.bfloat16)
    nl = (r1 - nm.astype(jnp.float32)).astype(jnp.bfloat16)

    ones = jnp.ones((_Q, 3), jnp.bfloat16)
    lhs = jnp.concatenate([xh, xm, xl, xh, xm, xh, ones], axis=1)
    rhs = jnp.concatenate([yh, yh, yh, ym, ym, yl, nh, nm, nl], axis=1)

    lane = lax.broadcasted_iota(jnp.int32, (_Q, _CS), 1)
    run_v = jnp.zeros((_Q, _CS), jnp.float32)
    run_col = lane
    for c in range(_NCH):
        s = lax.dot_general(
            lhs, rhs[c * _CS:(c + 1) * _CS],
            (((1,), (1,)), ((), ())),
            preferred_element_type=jnp.float32,
        )
        if c == 0:
            run_v = s
        else:
            gt = s > run_v
            run_v = jnp.where(gt, s, run_v)
            run_col = jnp.where(gt, lane + c * _CS, run_col)
    m = jnp.max(run_v, axis=1, keepdims=True)
    cand = jnp.where(run_v == m, run_col, jnp.int32(_S))
    idx = jnp.min(cand, axis=1)
    idx_ref[0, 0, :] = idx + b * _S


_sc_mesh = plsc.VectorSubcoreMesh(
    core_axis_name="c", subcore_axis_name="s",
    num_cores=_NC, num_subcores=_NS,
)


@functools.partial(
    pl.kernel,
    out_type=jax.ShapeDtypeStruct((_ROWS, 3 * _D), jnp.float32),
    mesh=_sc_mesh,
    scratch_types=[
        pltpu.VMEM((_RPW,), jnp.int32),
        pltpu.VMEM((_RPW, _D), jnp.float32),
        pltpu.VMEM((_RPW, _D), jnp.float32),
        pltpu.VMEM((_RPW, _D), jnp.float32),
        pltpu.VMEM((_RPW, 3 * _D), jnp.float32),
        pltpu.VMEM((_D,), jnp.float32),
        pltpu.VMEM((_D,), jnp.float32),
        pltpu.SemaphoreType.DMA,
    ],
    compiler_params=pltpu.CompilerParams(use_tc_tiling_on_sc=False),
)
def _sc_kernel(x0_hbm, data_hbm, t_hbm, mask_hbm, idx_hbm, out_hbm,
               idx_v, x0_v, dat_v, near_v, out_v, t_v, m_v, sem):
    wid = lax.axis_index("s") * _NC + lax.axis_index("c")
    base = wid * _RPW
    bidx = wid // (_NW // _B)
    pltpu.sync_copy(idx_hbm.at[pl.ds(base, _RPW)], idx_v)
    copies = [
        pltpu.async_copy(
            data_hbm.at[idx_v.at[pl.ds(j * _CHUNK, _CHUNK)]],
            near_v.at[pl.ds(j * _CHUNK, _CHUNK)],
            sem,
        )
        for j in range(_RPW // _CHUNK)
    ]
    pltpu.sync_copy(x0_hbm.at[pl.ds(base, _RPW)], x0_v)
    pltpu.sync_copy(data_hbm.at[pl.ds(base, _RPW)], dat_v)
    pltpu.sync_copy(t_hbm.at[bidx], t_v)
    pltpu.sync_copy(mask_hbm.at[0], m_v)
    for cpy in copies:
        cpy.wait()

    tv = t_v[...]
    mv = m_v[...] > 0.0
    omt = 1.0 - tv

    def body(i, carry):
        d = dat_v[i, :]
        x0r = x0_v[i, :]
        xmg = jnp.where(mv, d, x0r)
        out_v[i, pl.ds(0, _D)] = jnp.where(mv, d, tv * d + omt * xmg)
        out_v[i, pl.ds(_D, _D)] = d - xmg
        out_v[i, pl.ds(2 * _D, _D)] = near_v[i, :]
        return carry

    lax.fori_loop(0, _RPW, body, 0)
    pltpu.sync_copy(out_v, out_hbm.at[pl.ds(base, _RPW)])


def _split3(v):
    hi = v.astype(jnp.bfloat16)
    r1 = v - hi.astype(jnp.float32)
    mid = r1.astype(jnp.bfloat16)
    lo = (r1 - mid.astype(jnp.float32)).astype(jnp.bfloat16)
    return hi, mid, lo


@jax.jit
def kernel(x0, data, t, condition_mask):
    mask_f = condition_mask.reshape(1, _D).astype(jnp.float32)
    idxg = pl.pallas_call(
        _fm_kernel,
        grid=(_B,),
        in_specs=[
            pl.BlockSpec((1, _D), lambda b: (0, 0)),
            pl.BlockSpec((1, _S, _D), lambda b: (b, 0, 0)),
            pl.BlockSpec((1, _Q, _D), lambda b: (b, 0, 0)),
        ],
        out_specs=pl.BlockSpec((1, 1, _Q), lambda b: (b, 0, 0)),
        out_shape=jax.ShapeDtypeStruct((_B, 1, _S), jnp.int32),
    )(mask_f, data, x0)
    t4 = jnp.broadcast_to(t.reshape(_B, 1), (_B, _D)).astype(jnp.float32)
    out = _sc_kernel(
        x0.reshape(_ROWS, _D),
        data.reshape(_ROWS, _D),
        t4,
        mask_f,
        idxg.reshape(_ROWS),
    )
    return out.reshape(_B, _S, 3 * _D)
